# Initial kernel scaffold; baseline (speedup 1.0000x reference)
import jax
import jax.numpy as jnp
from jax.experimental import pallas as pl


def kernel(input):
    raise NotImplementedError("write your pallas kernel here")


# speedup vs baseline: 1.2098x; 1.2098x over previous
import jax
import jax.numpy as jnp
from jax import lax
from jax.experimental import pallas as pl
from jax.experimental.pallas import tpu as pltpu
from jax.experimental.pallas import tpu_sc as plsc

NUM_TOKENS = 32768
N_EXP = 64
NC = 2
NS = 16
NW = NC * NS
ROWS_PER_W = NUM_TOKENS // NW
WORDS_PER_W = ROWS_PER_W * N_EXP
GROUPS = ROWS_PER_W // 16

_KS0 = 0
_KS1 = 42
_KS2 = _KS1 ^ 0x1BD11BDA
_ROT_A = (13, 15, 26, 6)
_ROT_B = (17, 29, 16, 24)


def _rotl(x, r):
    return lax.shift_left(x, jnp.int32(r)) | lax.shift_right_logical(
        x, jnp.int32(32 - r))


def _threefry_bits23(j):
    x0 = jnp.zeros((16,), jnp.int32) + jnp.int32(_KS0)
    x1 = j + jnp.int32(_KS1)
    ks = (jnp.int32(_KS0), jnp.int32(_KS1), jnp.int32(_KS2))
    for g in range(5):
        rots = _ROT_A if g % 2 == 0 else _ROT_B
        for r in rots:
            x0 = x0 + x1
            x1 = _rotl(x1, r) ^ x0
        x0 = x0 + ks[(g + 1) % 3]
        x1 = x1 + ks[(g + 2) % 3] + jnp.int32(g + 1)
    return lax.shift_right_logical(x0 ^ x1, jnp.int32(9))


def _gate_body(out_hbm, buf, bits):
    wid = lax.axis_index("s") * NC + lax.axis_index("c")
    row0 = wid * ROWS_PER_W
    lane = lax.iota(jnp.int32, 16)
    lane64 = lane * jnp.int32(N_EXP)
    neg1 = jnp.full((16,), -1, jnp.int32)
    one = jnp.float32(1.0)
    zero = jnp.float32(0.0)

    def group(g, carry):
        jbase = (row0 + 16 * g) * N_EXP

        def pass1(e, t):
            t63, t62, t61 = t
            b = _threefry_bits23(lane64 + (jbase + e))
            bits[pl.ds(e * 16, 16)] = b
            hi = jnp.maximum(t63, b)
            c1 = jnp.minimum(t63, b)
            mid = jnp.maximum(t62, c1)
            c2 = jnp.minimum(t62, c1)
            lo = jnp.maximum(t61, c2)
            return hi, mid, lo

        _, t62, t61 = lax.fori_loop(0, N_EXP, pass1, (neg1, neg1, neg1))
        strict = t62 > t61

        def pass2(e, carry2):
            b = bits[pl.ds(e * 16, 16)]
            sel = (b > t62) | ((b == t62) & strict)
            idx = lane64 + (g * (16 * N_EXP) + e)
            plsc.store_scatter(buf, [idx], jnp.where(sel, one, zero))
            return carry2

        lax.fori_loop(0, N_EXP, pass2, 0)
        return carry

    lax.fori_loop(0, GROUPS, group, 0)
    pltpu.sync_copy(buf, out_hbm.at[pl.ds(wid * WORDS_PER_W, WORDS_PER_W)])


_gate = pl.kernel(
    _gate_body,
    out_type=jax.ShapeDtypeStruct((NUM_TOKENS * N_EXP,), jnp.float32),
    mesh=plsc.VectorSubcoreMesh(core_axis_name="c", subcore_axis_name="s"),
    scratch_types=[
        pltpu.VMEM((WORDS_PER_W,), jnp.float32),
        pltpu.VMEM((N_EXP * 16,), jnp.int32),
    ],
    compiler_params=pltpu.CompilerParams(needs_layout_passes=False),
)


def kernel(input):
    assert input.shape[0] == NUM_TOKENS
    return _gate().reshape(NUM_TOKENS, N_EXP)


# speedup vs baseline: 1.2164x; 1.0055x over previous
import jax
import jax.numpy as jnp
from jax import lax
from jax.experimental import pallas as pl
from jax.experimental.pallas import tpu as pltpu
from jax.experimental.pallas import tpu_sc as plsc

NUM_TOKENS = 32768
N_EXP = 64
NC = 2
NS = 16
NW = NC * NS
ROWS_PER_W = NUM_TOKENS // NW
WORDS_PER_W = ROWS_PER_W * N_EXP
GROUPS = ROWS_PER_W // 16

_KS0 = 0
_KS1 = 42
_KS2 = _KS1 ^ 0x1BD11BDA
_ROT_A = (13, 15, 26, 6)
_ROT_B = (17, 29, 16, 24)


def _rotl(x, r):
    return lax.shift_left(x, jnp.int32(r)) | lax.shift_right_logical(
        x, jnp.int32(32 - r))


def _threefry_bits23(j):
    x1 = j + jnp.int32(_KS1)
    x0 = x1
    x1 = _rotl(x1, _ROT_A[0]) ^ x0
    ks = (jnp.int32(_KS0), jnp.int32(_KS1), jnp.int32(_KS2))
    first = True
    for g in range(5):
        rots = _ROT_A if g % 2 == 0 else _ROT_B
        for r in (rots[1:] if first else rots):
            x0 = x0 + x1
            x1 = _rotl(x1, r) ^ x0
        first = False
        x0 = x0 + ks[(g + 1) % 3]
        x1 = x1 + ks[(g + 2) % 3] + jnp.int32(g + 1)
    return lax.shift_right_logical(x0 ^ x1, jnp.int32(9))


def _gate_body(out_hbm, buf, bits):
    wid = lax.axis_index("s") * NC + lax.axis_index("c")
    row0 = wid * ROWS_PER_W
    lane = lax.iota(jnp.int32, 16)
    lane64 = lane * jnp.int32(N_EXP)
    neg1 = jnp.full((16,), -1, jnp.int32)
    one = jnp.float32(1.0)
    zero = jnp.float32(0.0)

    UNROLL = 8

    def group(g, carry):
        jbase = (row0 + 16 * g) * N_EXP

        def pass1(i, t):
            t63, t62, t61 = t
            e0 = i * UNROLL
            bs = [_threefry_bits23(lane64 + (jbase + e0 + k))
                  for k in range(UNROLL)]
            for k, b in enumerate(bs):
                bits[pl.ds((e0 + k) * 16, 16)] = b
                hi = jnp.maximum(t63, b)
                c1 = jnp.minimum(t63, b)
                mid = jnp.maximum(t62, c1)
                c2 = jnp.minimum(t62, c1)
                lo = jnp.maximum(t61, c2)
                t63, t62, t61 = hi, mid, lo
            return t63, t62, t61

        _, t62, t61 = lax.fori_loop(0, N_EXP // UNROLL, pass1,
                                    (neg1, neg1, neg1))
        strict = t62 > t61

        def pass2(i, carry2):
            e0 = i * UNROLL
            for k in range(UNROLL):
                b = bits[pl.ds((e0 + k) * 16, 16)]
                sel = (b > t62) | ((b == t62) & strict)
                idx = lane64 + (g * (16 * N_EXP) + e0 + k)
                plsc.store_scatter(buf, [idx], jnp.where(sel, one, zero))
            return carry2

        lax.fori_loop(0, N_EXP // UNROLL, pass2, 0)
        return carry

    lax.fori_loop(0, GROUPS, group, 0)
    pltpu.sync_copy(buf, out_hbm.at[pl.ds(wid * WORDS_PER_W, WORDS_PER_W)])


_gate = pl.kernel(
    _gate_body,
    out_type=jax.ShapeDtypeStruct((NUM_TOKENS * N_EXP,), jnp.float32),
    mesh=plsc.VectorSubcoreMesh(core_axis_name="c", subcore_axis_name="s"),
    scratch_types=[
        pltpu.VMEM((WORDS_PER_W,), jnp.float32),
        pltpu.VMEM((N_EXP * 16,), jnp.int32),
    ],
    compiler_params=pltpu.CompilerParams(needs_layout_passes=False),
)


def kernel(input):
    assert input.shape[0] == NUM_TOKENS
    return _gate().reshape(NUM_TOKENS, N_EXP)


# speedup vs baseline: 1.8512x; 1.5218x over previous
import functools

import jax
import jax.numpy as jnp
from jax import lax
from jax.experimental import pallas as pl
from jax.experimental.pallas import tpu as pltpu
from jax.experimental.pallas import tpu_sc as plsc

NUM_TOKENS = 32768
N_EXP = 64
NC = 2
NS = 16
NW = NC * NS

SPLIT = 16384

_KS0 = 0
_KS1 = 42
_KS2 = _KS1 ^ 0x1BD11BDA
_ROT_A = (13, 15, 26, 6)
_ROT_B = (17, 29, 16, 24)


def _rotl(x, r):
    return lax.shift_left(x, jnp.int32(r)) | lax.shift_right_logical(
        x, jnp.int32(32 - r))


def _threefry_bits23(j):
    x1 = j + jnp.int32(_KS1)
    x0 = x1
    x1 = _rotl(x1, _ROT_A[0]) ^ x0
    ks = (jnp.int32(_KS0), jnp.int32(_KS1), jnp.int32(_KS2))
    first = True
    for g in range(5):
        rots = _ROT_A if g % 2 == 0 else _ROT_B
        for r in (rots[1:] if first else rots):
            x0 = x0 + x1
            x1 = _rotl(x1, r) ^ x0
        first = False
        x0 = x0 + ks[(g + 1) % 3]
        x1 = x1 + ks[(g + 2) % 3] + jnp.int32(g + 1)
    return lax.shift_right_logical(x0 ^ x1, jnp.int32(9))


SC_ROWS = SPLIT
SC_ROWS_PER_W = SC_ROWS // NW
SC_WORDS_PER_W = SC_ROWS_PER_W * N_EXP
SC_GROUPS = SC_ROWS_PER_W // 16
_UNROLL = 8


def _sc_body(out_hbm, buf, bits):
    wid = lax.axis_index("s") * NC + lax.axis_index("c")
    row0 = wid * SC_ROWS_PER_W
    lane = lax.iota(jnp.int32, 16)
    lane64 = lane * jnp.int32(N_EXP)
    neg1 = jnp.full((16,), -1, jnp.int32)
    one = jnp.float32(1.0)
    zero = jnp.float32(0.0)

    def group(g, carry):
        jbase = (row0 + 16 * g) * N_EXP

        def pass1(i, t):
            t63, t62, t61 = t
            e0 = i * _UNROLL
            bs = [_threefry_bits23(lane64 + (jbase + e0 + k))
                  for k in range(_UNROLL)]
            for k, b in enumerate(bs):
                bits[pl.ds((e0 + k) * 16, 16)] = b
                hi = jnp.maximum(t63, b)
                c1 = jnp.minimum(t63, b)
                mid = jnp.maximum(t62, c1)
                c2 = jnp.minimum(t62, c1)
                lo = jnp.maximum(t61, c2)
                t63, t62, t61 = hi, mid, lo
            return t63, t62, t61

        _, t62, t61 = lax.fori_loop(0, N_EXP // _UNROLL, pass1,
                                    (neg1, neg1, neg1))
        strict = t62 > t61

        def pass2(i, carry2):
            e0 = i * _UNROLL
            for k in range(_UNROLL):
                b = bits[pl.ds((e0 + k) * 16, 16)]
                sel = (b > t62) | ((b == t62) & strict)
                idx = lane64 + (g * (16 * N_EXP) + e0 + k)
                plsc.store_scatter(buf, [idx], jnp.where(sel, one, zero))
            return carry2

        lax.fori_loop(0, N_EXP // _UNROLL, pass2, 0)
        return carry

    lax.fori_loop(0, SC_GROUPS, group, 0)
    pltpu.sync_copy(buf, out_hbm.at[pl.ds(wid * SC_WORDS_PER_W,
                                          SC_WORDS_PER_W)])


@functools.cache
def _sc_gate_fn():
    return pl.kernel(
        _sc_body,
        out_type=jax.ShapeDtypeStruct((SC_ROWS * N_EXP,), jnp.float32),
        mesh=plsc.VectorSubcoreMesh(core_axis_name="c", subcore_axis_name="s"),
        scratch_types=[
            pltpu.VMEM((SC_WORDS_PER_W,), jnp.float32),
            pltpu.VMEM((N_EXP * 16,), jnp.int32),
        ],
        compiler_params=pltpu.CompilerParams(needs_layout_passes=False),
    )


TC_ROWS = NUM_TOKENS - SPLIT
TC_BLOCK = 1024
TC_R2 = TC_BLOCK // 2


def _tc_kernel(out_ref):
    i = pl.program_id(0)
    row0 = jnp.int32(SPLIT) + i * jnp.int32(TC_BLOCK)
    j = (row0 * jnp.int32(N_EXP)
         + lax.broadcasted_iota(jnp.int32, (N_EXP, TC_BLOCK), 1)
         * jnp.int32(N_EXP)
         + lax.broadcasted_iota(jnp.int32, (N_EXP, TC_BLOCK), 0))
    b = _threefry_bits23(j)
    m1 = jnp.max(b, axis=0, keepdims=True)
    e1 = b == m1
    z = jnp.where(e1, jnp.int32(-1), b)
    m2 = jnp.max(z, axis=0, keepdims=True)
    e2 = z == m2
    y = jnp.where(e2, jnp.int32(-1), z)
    m3 = jnp.max(y, axis=0, keepdims=True)
    c1 = jnp.sum(e1.astype(jnp.int32), axis=0, keepdims=True)
    c2 = jnp.sum(e2.astype(jnp.int32), axis=0, keepdims=True)
    t62 = jnp.where(c1 >= 2, m1, m2)
    t61 = jnp.where(c1 >= 3, m1, jnp.where(c1 + c2 >= 3, m2, m3))
    sel = (b > t62) | ((b == t62) & (t62 > t61))
    mask = jnp.where(sel, jnp.float32(1.0), jnp.float32(0.0))
    out_ref[...] = mask.T


if TC_ROWS > 0:
    _tc_gate = pl.pallas_call(
        _tc_kernel,
        out_shape=jax.ShapeDtypeStruct((TC_ROWS, N_EXP), jnp.float32),
        grid=(TC_ROWS // TC_BLOCK,),
        out_specs=pl.BlockSpec((TC_BLOCK, N_EXP), lambda i: (i, 0)),
        compiler_params=pltpu.CompilerParams(
            dimension_semantics=("arbitrary",)),
    )


def kernel(input):
    assert input.shape[0] == NUM_TOKENS
    parts = []
    if SPLIT > 0:
        parts.append(_sc_gate_fn()().reshape(SC_ROWS, N_EXP))
    if TC_ROWS > 0:
        parts.append(_tc_gate())
    if len(parts) == 1:
        return parts[0]
    return jnp.concatenate(parts, axis=0)


# speedup vs baseline: 3.9995x; 2.1605x over previous
import functools

import jax
import jax.numpy as jnp
from jax import lax
from jax.experimental import pallas as pl
from jax.experimental.pallas import tpu as pltpu
from jax.experimental.pallas import tpu_sc as plsc

NUM_TOKENS = 32768
N_EXP = 64
NC = 2
NS = 16
NW = NC * NS

SPLIT = 0

_KS0 = 0
_KS1 = 42
_KS2 = _KS1 ^ 0x1BD11BDA
_ROT_A = (13, 15, 26, 6)
_ROT_B = (17, 29, 16, 24)


def _rotl(x, r):
    return lax.shift_left(x, jnp.int32(r)) | lax.shift_right_logical(
        x, jnp.int32(32 - r))


def _threefry_bits23(j):
    x1 = j + jnp.int32(_KS1)
    x0 = x1
    x1 = _rotl(x1, _ROT_A[0]) ^ x0
    ks = (jnp.int32(_KS0), jnp.int32(_KS1), jnp.int32(_KS2))
    first = True
    for g in range(5):
        rots = _ROT_A if g % 2 == 0 else _ROT_B
        for r in (rots[1:] if first else rots):
            x0 = x0 + x1
            x1 = _rotl(x1, r) ^ x0
        first = False
        x0 = x0 + ks[(g + 1) % 3]
        x1 = x1 + ks[(g + 2) % 3] + jnp.int32(g + 1)
    return lax.shift_right_logical(x0 ^ x1, jnp.int32(9))


SC_ROWS = SPLIT
SC_ROWS_PER_W = SC_ROWS // NW
SC_WORDS_PER_W = SC_ROWS_PER_W * N_EXP
SC_GROUPS = SC_ROWS_PER_W // 16
_UNROLL = 8


def _sc_body(out_hbm, buf, bits):
    wid = lax.axis_index("s") * NC + lax.axis_index("c")
    row0 = wid * SC_ROWS_PER_W
    lane = lax.iota(jnp.int32, 16)
    lane64 = lane * jnp.int32(N_EXP)
    neg1 = jnp.full((16,), -1, jnp.int32)
    one = jnp.float32(1.0)
    zero = jnp.float32(0.0)

    def group(g, carry):
        jbase = (row0 + 16 * g) * N_EXP

        def pass1(i, t):
            t63, t62, t61 = t
            e0 = i * _UNROLL
            bs = [_threefry_bits23(lane64 + (jbase + e0 + k))
                  for k in range(_UNROLL)]
            for k, b in enumerate(bs):
                bits[pl.ds((e0 + k) * 16, 16)] = b
                hi = jnp.maximum(t63, b)
                c1 = jnp.minimum(t63, b)
                mid = jnp.maximum(t62, c1)
                c2 = jnp.minimum(t62, c1)
                lo = jnp.maximum(t61, c2)
                t63, t62, t61 = hi, mid, lo
            return t63, t62, t61

        _, t62, t61 = lax.fori_loop(0, N_EXP // _UNROLL, pass1,
                                    (neg1, neg1, neg1))
        strict = t62 > t61

        def pass2(i, carry2):
            e0 = i * _UNROLL
            for k in range(_UNROLL):
                b = bits[pl.ds((e0 + k) * 16, 16)]
                sel = (b > t62) | ((b == t62) & strict)
                idx = lane64 + (g * (16 * N_EXP) + e0 + k)
                plsc.store_scatter(buf, [idx], jnp.where(sel, one, zero))
            return carry2

        lax.fori_loop(0, N_EXP // _UNROLL, pass2, 0)
        return carry

    lax.fori_loop(0, SC_GROUPS, group, 0)
    pltpu.sync_copy(buf, out_hbm.at[pl.ds(wid * SC_WORDS_PER_W,
                                          SC_WORDS_PER_W)])


@functools.cache
def _sc_gate_fn():
    return pl.kernel(
        _sc_body,
        out_type=jax.ShapeDtypeStruct((SC_ROWS * N_EXP,), jnp.float32),
        mesh=plsc.VectorSubcoreMesh(core_axis_name="c", subcore_axis_name="s"),
        scratch_types=[
            pltpu.VMEM((SC_WORDS_PER_W,), jnp.float32),
            pltpu.VMEM((N_EXP * 16,), jnp.int32),
        ],
        compiler_params=pltpu.CompilerParams(needs_layout_passes=False),
    )


TC_ROWS = NUM_TOKENS - SPLIT
TC_BLOCK = 1024
TC_R2 = TC_BLOCK // 2


def _tc_kernel(out_ref):
    i = pl.program_id(0)
    row0 = jnp.int32(SPLIT) + i * jnp.int32(TC_BLOCK)
    j = (row0 * jnp.int32(N_EXP)
         + lax.broadcasted_iota(jnp.int32, (N_EXP, TC_BLOCK), 1)
         * jnp.int32(N_EXP)
         + lax.broadcasted_iota(jnp.int32, (N_EXP, TC_BLOCK), 0))
    b = _threefry_bits23(j)
    m1 = jnp.max(b, axis=0, keepdims=True)
    e1 = b == m1
    z = jnp.where(e1, jnp.int32(-1), b)
    m2 = jnp.max(z, axis=0, keepdims=True)
    e2 = z == m2
    y = jnp.where(e2, jnp.int32(-1), z)
    m3 = jnp.max(y, axis=0, keepdims=True)
    c1 = jnp.sum(e1.astype(jnp.int32), axis=0, keepdims=True)
    c2 = jnp.sum(e2.astype(jnp.int32), axis=0, keepdims=True)
    t62 = jnp.where(c1 >= 2, m1, m2)
    t61 = jnp.where(c1 >= 3, m1, jnp.where(c1 + c2 >= 3, m2, m3))
    sel = (b > t62) | ((b == t62) & (t62 > t61))
    mask = jnp.where(sel, jnp.float32(1.0), jnp.float32(0.0))
    out_ref[...] = mask.T


if TC_ROWS > 0:
    _tc_gate = pl.pallas_call(
        _tc_kernel,
        out_shape=jax.ShapeDtypeStruct((TC_ROWS, N_EXP), jnp.float32),
        grid=(TC_ROWS // TC_BLOCK,),
        out_specs=pl.BlockSpec((TC_BLOCK, N_EXP), lambda i: (i, 0)),
        compiler_params=pltpu.CompilerParams(
            dimension_semantics=("arbitrary",)),
    )


def kernel(input):
    assert input.shape[0] == NUM_TOKENS
    parts = []
    if SPLIT > 0:
        parts.append(_sc_gate_fn()().reshape(SC_ROWS, N_EXP))
    if TC_ROWS > 0:
        parts.append(_tc_gate())
    if len(parts) == 1:
        return parts[0]
    return jnp.concatenate(parts, axis=0)
